# Initial kernel scaffold; baseline (speedup 1.0000x reference)
import jax
import jax.numpy as jnp
from jax.experimental import pallas as pl


def kernel(x, edge_index, W1a, b1a, W1b, b1b, W2a, b2a, W2b, b2b, Wc, bc):
    raise NotImplementedError("write your pallas kernel here")


# speedup vs baseline: 3.1404x; 3.1404x over previous
import jax
import jax.numpy as jnp
from jax import lax
from jax.experimental import pallas as pl
from jax.experimental.pallas import tpu as pltpu
from jax.experimental.pallas import tpu_sc as plsc

N_NODES = 10000
D = 128
N_CORES = 2
N_SUB = 16
N_WORK = N_CORES * N_SUB
CHUNK = 128
ROWS_PER_TILE = 624
TAIL_BASE = ROWS_PER_TILE * N_SUB
TAIL_ROWS = N_NODES - TAIL_BASE
TRASH = N_NODES
ACC_ROWS = N_NODES + 8


def _sc_agg_body(feats, srcs, dsts, zeros, out, idx_s, idx_d, rows, acc, sem):
    c = lax.axis_index("c")
    s = lax.axis_index("s")
    w = c * N_SUB + s
    rpw = idx_s.shape[0]

    base = s * ROWS_PER_TILE

    def init_from(ref):
        pltpu.sync_copy(ref.at[pl.ds(base, ROWS_PER_TILE)],
                        acc.at[pl.ds(base, ROWS_PER_TILE)])

        @pl.when(s == N_SUB - 1)
        def _():
            pltpu.sync_copy(ref.at[pl.ds(TAIL_BASE, TAIL_ROWS)],
                            acc.at[pl.ds(TAIL_BASE, TAIL_ROWS)])

    @pl.when(c == 0)
    def _():
        init_from(feats)

    @pl.when(c != 0)
    def _():
        init_from(zeros)

    pltpu.sync_copy(srcs.at[pl.ds(w * rpw, rpw)], idx_s)
    pltpu.sync_copy(dsts.at[pl.ds(w * rpw, rpw)], idx_d)
    plsc.subcore_barrier()

    def step(k, carry):
        pltpu.async_copy(feats.at[idx_s.at[k]], rows, sem).wait()
        pltpu.sync_copy(rows, acc.at[idx_d.at[k]], add=True)
        return carry

    lax.fori_loop(0, rpw, step, 0)

    plsc.subcore_barrier()
    pltpu.sync_copy(acc.at[pl.ds(base, ROWS_PER_TILE)],
                    out.at[c, pl.ds(base, ROWS_PER_TILE)])

    @pl.when(s == N_SUB - 1)
    def _():
        pltpu.sync_copy(acc.at[pl.ds(TAIL_BASE, TAIL_ROWS)],
                        out.at[c, pl.ds(TAIL_BASE, TAIL_ROWS)])


def _sc_agg(feats, srcs, dsts, zeros):
    rpw = srcs.shape[0] // N_WORK
    fn = pl.kernel(
        _sc_agg_body,
        out_type=jax.ShapeDtypeStruct((N_CORES, N_NODES, D), jnp.float32),
        mesh=plsc.VectorSubcoreMesh(core_axis_name="c", subcore_axis_name="s",
                                    num_cores=N_CORES, num_subcores=N_SUB),
        scratch_types=[
            pltpu.VMEM((rpw, CHUNK), jnp.int32),
            pltpu.VMEM((rpw, CHUNK), jnp.int32),
            pltpu.VMEM((CHUNK, D), jnp.float32),
            pltpu.VMEM_SHARED((ACC_ROWS, D), jnp.float32),
            pltpu.SemaphoreType.DMA,
        ],
    )
    return fn(feats, srcs, dsts, zeros)


ROW_BLK = 2000


def _mlp_body(p_ref, wa, ba, wb, bb, out_ref):
    h = p_ref[0] + p_ref[1]
    t = jnp.maximum(jnp.dot(h, wa[...], preferred_element_type=jnp.float32) + ba[...], 0.0)
    out_ref[...] = jnp.dot(t, wb[...], preferred_element_type=jnp.float32) + bb[...]


def _mlp(p, Wa, ba, Wb, bb):
    return pl.pallas_call(
        _mlp_body,
        grid=(N_NODES // ROW_BLK,),
        in_specs=[
            pl.BlockSpec((N_CORES, ROW_BLK, D), lambda i: (0, i, 0)),
            pl.BlockSpec((D, D), lambda i: (0, 0)),
            pl.BlockSpec((1, D), lambda i: (0, 0)),
            pl.BlockSpec((D, D), lambda i: (0, 0)),
            pl.BlockSpec((1, D), lambda i: (0, 0)),
        ],
        out_specs=pl.BlockSpec((ROW_BLK, D), lambda i: (i, 0)),
        out_shape=jax.ShapeDtypeStruct((N_NODES, D), jnp.float32),
    )(p, Wa, ba.reshape(1, D), Wb, bb.reshape(1, D))


def _mlp_pool_body(p_ref, wa, ba, wb, bb, wc, bcp, out_ref, acc):
    i = pl.program_id(0)

    @pl.when(i == 0)
    def _():
        acc[...] = jnp.zeros_like(acc)

    h = p_ref[0] + p_ref[1]
    t = jnp.maximum(jnp.dot(h, wa[...], preferred_element_type=jnp.float32) + ba[...], 0.0)
    h2 = jnp.dot(t, wb[...], preferred_element_type=jnp.float32) + bb[...]
    acc[...] += jnp.sum(h2, axis=0, keepdims=True)

    @pl.when(i == pl.num_programs(0) - 1)
    def _():
        out_ref[...] = jnp.dot(acc[...] * (1.0 / N_NODES), wc[...],
                               preferred_element_type=jnp.float32) + bcp[...]


def _mlp_pool(p, Wa, ba, Wb, bb, Wcp, bcp):
    return pl.pallas_call(
        _mlp_pool_body,
        grid=(N_NODES // ROW_BLK,),
        in_specs=[
            pl.BlockSpec((N_CORES, ROW_BLK, D), lambda i: (0, i, 0)),
            pl.BlockSpec((D, D), lambda i: (0, 0)),
            pl.BlockSpec((1, D), lambda i: (0, 0)),
            pl.BlockSpec((D, D), lambda i: (0, 0)),
            pl.BlockSpec((1, D), lambda i: (0, 0)),
            pl.BlockSpec((D, D), lambda i: (0, 0)),
            pl.BlockSpec((1, D), lambda i: (0, 0)),
        ],
        out_specs=pl.BlockSpec((1, D), lambda i: (0, 0)),
        out_shape=jax.ShapeDtypeStruct((1, D), jnp.float32),
        scratch_shapes=[pltpu.VMEM((1, D), jnp.float32)],
    )(p, Wa, ba.reshape(1, D), Wb, bb.reshape(1, D), Wcp, bcp)


def kernel(x, edge_index, W1a, b1a, W1b, b1b, W2a, b2a, W2b, b2b, Wc, bc):
    ei = edge_index.astype(jnp.int32)
    src, dst = ei[0], ei[1]
    n_edges = src.shape[0]
    rpw = -(-n_edges // (N_WORK * CHUNK))
    rpw = -(-rpw // 8) * 8
    pad = N_WORK * rpw * CHUNK - n_edges
    src_p = jnp.concatenate([src, jnp.zeros((pad,), jnp.int32)]).reshape(N_WORK * rpw, CHUNK)
    dst_p = jnp.concatenate([dst, jnp.full((pad,), TRASH, jnp.int32)]).reshape(N_WORK * rpw, CHUNK)
    zeros = jnp.zeros((N_NODES, D), jnp.float32)

    p1 = _sc_agg(x, src_p, dst_p, zeros)
    h1 = _mlp(p1, W1a, b1a, W1b, b1b)
    p2 = _sc_agg(h1, src_p, dst_p, zeros)

    n_cls = Wc.shape[1]
    Wcp = jnp.pad(Wc, ((0, 0), (0, D - n_cls)))
    bcp = jnp.pad(bc, (0, D - n_cls)).reshape(1, D)
    out = _mlp_pool(p2, W2a, b2a, W2b, b2b, Wcp, bcp)
    return out[:, :n_cls]


# speedup vs baseline: 3.1431x; 1.0009x over previous
import jax
import jax.numpy as jnp
from jax import lax
from jax.experimental import pallas as pl
from jax.experimental.pallas import tpu as pltpu
from jax.experimental.pallas import tpu_sc as plsc

N_NODES = 10000
D = 128
N_CORES = 2
N_SUB = 16
N_WORK = N_CORES * N_SUB
CHUNK = 128
ROWS_PER_TILE = 624
TAIL_BASE = ROWS_PER_TILE * N_SUB
TAIL_ROWS = N_NODES - TAIL_BASE
TRASH_ROWS = 1024
ACC_ROWS = N_NODES + TRASH_ROWS


def _sc_agg_body(feats, srcs, dsts, zeros, out, idx_s, idx_d, rows, acc, sem):
    c = lax.axis_index("c")
    s = lax.axis_index("s")
    w = c * N_SUB + s
    rpw = idx_s.shape[0]

    base = s * ROWS_PER_TILE

    def init_from(ref):
        pltpu.sync_copy(ref.at[pl.ds(base, ROWS_PER_TILE)],
                        acc.at[pl.ds(base, ROWS_PER_TILE)])

        @pl.when(s == N_SUB - 1)
        def _():
            pltpu.sync_copy(ref.at[pl.ds(TAIL_BASE, TAIL_ROWS)],
                            acc.at[pl.ds(TAIL_BASE, TAIL_ROWS)])

    @pl.when(c == 0)
    def _():
        init_from(feats)

    @pl.when(c != 0)
    def _():
        init_from(zeros)

    pltpu.sync_copy(srcs.at[pl.ds(w * rpw, rpw)], idx_s)
    pltpu.sync_copy(dsts.at[pl.ds(w * rpw, rpw)], idx_d)
    plsc.subcore_barrier()

    def step(k, carry):
        pltpu.async_copy(feats.at[idx_s.at[k]], rows, sem).wait()
        pltpu.sync_copy(rows, acc.at[idx_d.at[k]], add=True)
        return carry

    lax.fori_loop(0, rpw, step, 0)

    plsc.subcore_barrier()
    pltpu.sync_copy(acc.at[pl.ds(base, ROWS_PER_TILE)],
                    out.at[c, pl.ds(base, ROWS_PER_TILE)])

    @pl.when(s == N_SUB - 1)
    def _():
        pltpu.sync_copy(acc.at[pl.ds(TAIL_BASE, TAIL_ROWS)],
                        out.at[c, pl.ds(TAIL_BASE, TAIL_ROWS)])


def _sc_agg(feats, srcs, dsts, zeros):
    rpw = srcs.shape[0] // N_WORK
    fn = pl.kernel(
        _sc_agg_body,
        out_type=jax.ShapeDtypeStruct((N_CORES, N_NODES, D), jnp.float32),
        mesh=plsc.VectorSubcoreMesh(core_axis_name="c", subcore_axis_name="s",
                                    num_cores=N_CORES, num_subcores=N_SUB),
        scratch_types=[
            pltpu.VMEM((rpw, CHUNK), jnp.int32),
            pltpu.VMEM((rpw, CHUNK), jnp.int32),
            pltpu.VMEM((CHUNK, D), jnp.float32),
            pltpu.VMEM_SHARED((ACC_ROWS, D), jnp.float32),
            pltpu.SemaphoreType.DMA,
        ],
    )
    return fn(feats, srcs, dsts, zeros)


ROW_BLK = 2000


def _mlp_body(p_ref, wa, ba, wb, bb, out_ref):
    h = p_ref[0] + p_ref[1]
    t = jnp.maximum(jnp.dot(h, wa[...], preferred_element_type=jnp.float32) + ba[...], 0.0)
    out_ref[...] = jnp.dot(t, wb[...], preferred_element_type=jnp.float32) + bb[...]


def _mlp(p, Wa, ba, Wb, bb):
    return pl.pallas_call(
        _mlp_body,
        grid=(N_NODES // ROW_BLK,),
        in_specs=[
            pl.BlockSpec((N_CORES, ROW_BLK, D), lambda i: (0, i, 0)),
            pl.BlockSpec((D, D), lambda i: (0, 0)),
            pl.BlockSpec((1, D), lambda i: (0, 0)),
            pl.BlockSpec((D, D), lambda i: (0, 0)),
            pl.BlockSpec((1, D), lambda i: (0, 0)),
        ],
        out_specs=pl.BlockSpec((ROW_BLK, D), lambda i: (i, 0)),
        out_shape=jax.ShapeDtypeStruct((N_NODES, D), jnp.float32),
    )(p, Wa, ba.reshape(1, D), Wb, bb.reshape(1, D))


def _mlp_pool_body(p_ref, wa, ba, wb, bb, wc, bcp, out_ref, acc):
    i = pl.program_id(0)

    @pl.when(i == 0)
    def _():
        acc[...] = jnp.zeros_like(acc)

    h = p_ref[0] + p_ref[1]
    t = jnp.maximum(jnp.dot(h, wa[...], preferred_element_type=jnp.float32) + ba[...], 0.0)
    h2 = jnp.dot(t, wb[...], preferred_element_type=jnp.float32) + bb[...]
    acc[...] += jnp.sum(h2, axis=0, keepdims=True)

    @pl.when(i == pl.num_programs(0) - 1)
    def _():
        out_ref[...] = jnp.dot(acc[...] * (1.0 / N_NODES), wc[...],
                               preferred_element_type=jnp.float32) + bcp[...]


def _mlp_pool(p, Wa, ba, Wb, bb, Wcp, bcp):
    return pl.pallas_call(
        _mlp_pool_body,
        grid=(N_NODES // ROW_BLK,),
        in_specs=[
            pl.BlockSpec((N_CORES, ROW_BLK, D), lambda i: (0, i, 0)),
            pl.BlockSpec((D, D), lambda i: (0, 0)),
            pl.BlockSpec((1, D), lambda i: (0, 0)),
            pl.BlockSpec((D, D), lambda i: (0, 0)),
            pl.BlockSpec((1, D), lambda i: (0, 0)),
            pl.BlockSpec((D, D), lambda i: (0, 0)),
            pl.BlockSpec((1, D), lambda i: (0, 0)),
        ],
        out_specs=pl.BlockSpec((1, D), lambda i: (0, 0)),
        out_shape=jax.ShapeDtypeStruct((1, D), jnp.float32),
        scratch_shapes=[pltpu.VMEM((1, D), jnp.float32)],
    )(p, Wa, ba.reshape(1, D), Wb, bb.reshape(1, D), Wcp, bcp)


def kernel(x, edge_index, W1a, b1a, W1b, b1b, W2a, b2a, W2b, b2b, Wc, bc):
    ei = edge_index.astype(jnp.int32)
    src, dst = ei[0], ei[1]
    n_edges = src.shape[0]
    rpw = -(-n_edges // (N_WORK * CHUNK))
    rpw = -(-rpw // 8) * 8
    pad = N_WORK * rpw * CHUNK - n_edges
    src_p = jnp.concatenate([src, jnp.zeros((pad,), jnp.int32)]).reshape(N_WORK * rpw, CHUNK)
    trash = N_NODES + jnp.arange(pad, dtype=jnp.int32) % TRASH_ROWS
    dst_p = jnp.concatenate([dst, trash]).reshape(N_WORK * rpw, CHUNK)
    zeros = jnp.zeros((N_NODES, D), jnp.float32)

    p1 = _sc_agg(x, src_p, dst_p, zeros)
    h1 = _mlp(p1, W1a, b1a, W1b, b1b)
    p2 = _sc_agg(h1, src_p, dst_p, zeros)

    n_cls = Wc.shape[1]
    Wcp = jnp.pad(Wc, ((0, 0), (0, D - n_cls)))
    bcp = jnp.pad(bc, (0, D - n_cls)).reshape(1, D)
    out = _mlp_pool(p2, W2a, b2a, W2b, b2b, Wcp, bcp)
    return out[:, :n_cls]
